# baseline (speedup 1.0000x reference)
import functools
import jax
import jax.numpy as jnp
from jax.experimental import pallas as pl
from jax.experimental.pallas import tpu as pltpu


def _k1_body(feat_ref, wfc_ref, al_ref, ar_ref, fc_ref, el_ref, er_ref):
    fc = jnp.dot(feat_ref[...], wfc_ref[...], preferred_element_type=jnp.float32)
    fc_ref[...] = fc
    el_ref[...] = jnp.dot(fc, al_ref[...], preferred_element_type=jnp.float32)
    er_ref[...] = jnp.dot(fc, ar_ref[...], preferred_element_type=jnp.float32)


def _k2_body(src_ref, dst_ref, fc_ref, el_ref, er_ref, bb_ref, rst_ref, denom_ref,
             *, ch, nchunks):
    p = pl.program_id(0)
    j = pl.program_id(1)

    @pl.when(jnp.logical_and(p == 0, j == 0))
    def _():
        denom_ref[...] = jnp.zeros_like(denom_ref)
        rst_ref[...] = jnp.zeros_like(rst_ref)

    @pl.when(jnp.logical_and(p == 1, j == 0))
    def _():
        denom_ref[...] = 1.0 / (denom_ref[...] + 1e-9)

    @pl.when(p == 0)
    def _():
        def body(i, _):
            s = src_ref[0, 0, i]
            d = dst_ref[0, 0, i]
            e = el_ref[pl.ds(s, 1), :] + er_ref[pl.ds(d, 1), :]
            e = jnp.where(e > 0, e, 0.2 * e)
            ex = jnp.exp(e)
            denom_ref[pl.ds(d, 1), :] += ex
            return 0
        jax.lax.fori_loop(0, ch, body, 0)

    @pl.when(p == 1)
    def _():
        def body(i, _):
            s = src_ref[0, 0, i]
            d = dst_ref[0, 0, i]
            e = el_ref[pl.ds(s, 1), :] + er_ref[pl.ds(d, 1), :]
            e = jnp.where(e > 0, e, 0.2 * e)
            ex = jnp.exp(e)
            alpha = ex * denom_ref[pl.ds(d, 1), :]
            am = jnp.dot(alpha, bb_ref[...], preferred_element_type=jnp.float32)
            rst_ref[pl.ds(d, 1), :] += am * fc_ref[pl.ds(s, 1), :]
            return 0
        jax.lax.fori_loop(0, ch, body, 0)


def _k3_body(rst_ref, bias_ref, w3_ref, b3_ref, out_ref):
    h = rst_ref[...] + bias_ref[...]
    h = jnp.where(h > 0, h, jnp.exp(jnp.minimum(h, 0.0)) - 1.0)
    out_ref[...] = jnp.dot(h, w3_ref[...], preferred_element_type=jnp.float32) + b3_ref[...]


def kernel(feat, edge_index, W_fc, attn_l, attn_r, bias_gat, W3, b3):
    N, IN_FEATS = feat.shape
    HH, HF = attn_l.shape
    D = HH * HF
    NUM_CLASSES = W3.shape[1]
    E = edge_index.shape[1]

    eye = jnp.eye(HH, dtype=jnp.float32)
    A_l = (attn_l[:, :, None] * eye[:, None, :]).reshape(D, HH)
    A_r = (attn_r[:, :, None] * eye[:, None, :]).reshape(D, HH)
    B_bcast = jnp.repeat(eye, HF, axis=1)

    RB = 1000 if N % 1000 == 0 else N
    n_rb = N // RB
    fc, el, er = pl.pallas_call(
        _k1_body,
        grid=(n_rb,),
        in_specs=[
            pl.BlockSpec((RB, IN_FEATS), lambda i: (i, 0)),
            pl.BlockSpec((IN_FEATS, D), lambda i: (0, 0)),
            pl.BlockSpec((D, HH), lambda i: (0, 0)),
            pl.BlockSpec((D, HH), lambda i: (0, 0)),
        ],
        out_specs=[
            pl.BlockSpec((RB, D), lambda i: (i, 0)),
            pl.BlockSpec((RB, HH), lambda i: (i, 0)),
            pl.BlockSpec((RB, HH), lambda i: (i, 0)),
        ],
        out_shape=[
            jax.ShapeDtypeStruct((N, D), jnp.float32),
            jax.ShapeDtypeStruct((N, HH), jnp.float32),
            jax.ShapeDtypeStruct((N, HH), jnp.float32),
        ],
    )(feat, W_fc, A_l, A_r)

    nchunks = 20 if E % 20 == 0 else 1
    ch = E // nchunks
    src = edge_index[0].reshape(nchunks, 1, ch)
    dst = edge_index[1].reshape(nchunks, 1, ch)

    rst = pl.pallas_call(
        functools.partial(_k2_body, ch=ch, nchunks=nchunks),
        grid=(2, nchunks),
        in_specs=[
            pl.BlockSpec((1, 1, ch), lambda p, j: (j, 0, 0), memory_space=pltpu.SMEM),
            pl.BlockSpec((1, 1, ch), lambda p, j: (j, 0, 0), memory_space=pltpu.SMEM),
            pl.BlockSpec((N, D), lambda p, j: (0, 0)),
            pl.BlockSpec((N, HH), lambda p, j: (0, 0)),
            pl.BlockSpec((N, HH), lambda p, j: (0, 0)),
            pl.BlockSpec((HH, D), lambda p, j: (0, 0)),
        ],
        out_specs=pl.BlockSpec((N, D), lambda p, j: (0, 0)),
        out_shape=jax.ShapeDtypeStruct((N, D), jnp.float32),
        scratch_shapes=[pltpu.VMEM((N, HH), jnp.float32)],
    )(src, dst, fc, el, er, B_bcast)

    out = pl.pallas_call(
        _k3_body,
        grid=(n_rb,),
        in_specs=[
            pl.BlockSpec((RB, D), lambda i: (i, 0)),
            pl.BlockSpec((1, D), lambda i: (0, 0)),
            pl.BlockSpec((D, NUM_CLASSES), lambda i: (0, 0)),
            pl.BlockSpec((1, NUM_CLASSES), lambda i: (0, 0)),
        ],
        out_specs=pl.BlockSpec((RB, NUM_CLASSES), lambda i: (i, 0)),
        out_shape=jax.ShapeDtypeStruct((N, NUM_CLASSES), jnp.float32),
    )(rst, bias_gat.reshape(1, D), W3, b3.reshape(1, NUM_CLASSES))

    return out


# speedup vs baseline: 13.1959x; 13.1959x over previous
import functools
import jax
import jax.numpy as jnp
from jax import lax
from jax.experimental import pallas as pl
from jax.experimental.pallas import tpu as pltpu
from jax.experimental.pallas import tpu_sc as plsc

G = 128
NSC = 2
NTILES = 16
LW = 128


def _k1_body(feat_ref, wfc_ref, al_ref, ar_ref, fcP_ref, el_ref, er_ref, *, NP):
    fc = jnp.dot(feat_ref[...], wfc_ref[...], preferred_element_type=jnp.float32)
    for p in range(NP):
        fcP_ref[p] = fc[:, p * LW:(p + 1) * LW]
    el_ref[...] = jnp.dot(fc, al_ref[...], preferred_element_type=jnp.float32)
    er_ref[...] = jnp.dot(fc, ar_ref[...], preferred_element_type=jnp.float32)


def _k3_body(rstP_ref, dpart_ref, bb_ref, bias_ref, w3_ref, b3_ref, out_ref, *, H, NP):
    d = dpart_ref[:, 0:H]
    invd = 1.0 / (d + 1e-9)
    invb = jnp.dot(invd, bb_ref[...], preferred_element_type=jnp.float32)
    rst = jnp.concatenate([rstP_ref[p] for p in range(NP)], axis=-1)
    hact = rst * invb + bias_ref[...]
    hact = jnp.where(hact > 0, hact, jnp.exp(jnp.minimum(hact, 0.0)) - 1.0)
    out_ref[...] = jnp.dot(hact, w3_ref[...], preferred_element_type=jnp.float32) + b3_ref[...]


def _make_s1(NV, NG):
    mesh = plsc.VectorSubcoreMesh(core_axis_name="c", subcore_axis_name="s")
    NVH = NV // 2
    GPT = NG // NTILES
    zrows = NVH // NTILES
    sch = zrows // 4

    @functools.partial(
        pl.kernel, mesh=mesh,
        out_type=[
            jax.ShapeDtypeStruct((NG * G * 16,), jnp.float32),
            jax.ShapeDtypeStruct((NV, 128), jnp.float32),
        ],
        scratch_types=[
            pltpu.VMEM((G,), jnp.int32),
            pltpu.VMEM((G,), jnp.int32),
            pltpu.VMEM((G,), jnp.int32),
            pltpu.VMEM((G, LW), jnp.float32),
            pltpu.VMEM((G, LW), jnp.float32),
            pltpu.VMEM((G, LW), jnp.float32),
            pltpu.VMEM((G * 16,), jnp.float32),
            pltpu.VMEM((NVH // NTILES // 4, LW), jnp.float32),
            pltpu.VMEM_SHARED((NVH + 16, LW), jnp.float32),
            pltpu.SemaphoreType.DMA,
            pltpu.SemaphoreType.DMA,
        ],
    )
    def s1(src_hbm, dst_hbm, el_hbm, er_hbm, z_hbm, eexp_hbm, denom_hbm,
           srcb, dstb, dstb2, rows_s, rows_d, exb, exf, stage, denom_sp,
           sem1, sem2):
        c = lax.axis_index("c")
        s = lax.axis_index("s")
        nbase = c * NVH
        pltpu.sync_copy(z_hbm, stage)
        for q in range(4):
            pltpu.sync_copy(stage, denom_sp.at[pl.ds(s * zrows + q * sch, sch)])
        plsc.subcore_barrier()

        def body(j, carry):
            g = s * GPT + j
            pltpu.sync_copy(src_hbm.at[pl.ds(g * G, G)], srcb)
            pltpu.sync_copy(dst_hbm.at[pl.ds(g * G, G)], dstb)
            cp1 = pltpu.async_copy(el_hbm.at[srcb], rows_s, sem1)
            cp2 = pltpu.async_copy(er_hbm.at[dstb], rows_d, sem2)
            cp1.wait()
            cp2.wait()
            for k in range(G // 16):
                dv = dstb[pl.ds(k * 16, 16)] - nbase
                ok = jnp.logical_and(dv >= 0, dv < NVH)
                dstb2[pl.ds(k * 16, 16)] = jnp.where(ok, dv, NVH)

            def inner(i, cc):
                v = rows_s[i, 0:16] + rows_d[i, 0:16]
                v = jnp.where(v > 0, v, 0.2 * v)
                ex = jnp.exp(v)
                exb[i, 0:16] = ex
                exf[pl.ds(i * 16, 16)] = ex
                return cc

            lax.fori_loop(0, G, inner, 0)

            @pl.when(c == 0)
            def _():
                pltpu.sync_copy(exf, eexp_hbm.at[pl.ds(g * G * 16, G * 16)])

            pltpu.sync_copy(exb, denom_sp.at[dstb2], add=True)
            return carry

        lax.fori_loop(0, GPT, body, 0)
        plsc.subcore_barrier()
        for q in range(4):
            base = s * zrows + q * sch
            pltpu.sync_copy(denom_sp.at[pl.ds(base, sch)], stage)
            pltpu.sync_copy(stage, denom_hbm.at[pl.ds(nbase + base, sch)])

    return s1


def _make_s2(NV, NG, GPT2, NP):
    mesh = plsc.VectorSubcoreMesh(core_axis_name="c", subcore_axis_name="s")
    zrows = NV // NTILES
    sch = zrows // 10
    ppc = NP // NSC

    @functools.partial(
        pl.kernel, mesh=mesh,
        out_type=jax.ShapeDtypeStruct((NP * NV, LW), jnp.float32),
        scratch_types=[
            pltpu.VMEM((G,), jnp.int32),
            pltpu.VMEM((G,), jnp.int32),
            pltpu.VMEM((G,), jnp.int32),
            pltpu.VMEM((G * 16,), jnp.float32),
            pltpu.VMEM((G, LW), jnp.float32),
            pltpu.VMEM((NV // NTILES // 10, LW), jnp.float32),
            pltpu.VMEM_SHARED((NV, LW), jnp.float32),
            pltpu.SemaphoreType.DMA,
        ],
    )
    def s2(srcP_hbm, dst_hbm, fcP_hbm, eexp_hbm, z128_hbm, rstP_hbm,
           srcb, dstb, idxb, exf, fcb, stage, rst_sp, sem):
        c = lax.axis_index("c")
        s = lax.axis_index("s")
        for pp in range(ppc):
            p = c * ppc + pp
            pltpu.sync_copy(z128_hbm, stage)
            for q in range(10):
                pltpu.sync_copy(
                    stage, rst_sp.at[pl.ds(s * zrows + q * sch, sch)])
            plsc.subcore_barrier()

            def body(j, carry):
                g = s * GPT2 + j
                pltpu.sync_copy(srcP_hbm.at[pl.ds(p * NG * G + g * G, G)], idxb)
                pltpu.sync_copy(dst_hbm.at[pl.ds(g * G, G)], dstb)
                pltpu.async_copy(fcP_hbm.at[idxb], fcb, sem).wait()
                pltpu.sync_copy(eexp_hbm.at[pl.ds(g * G * 16, G * 16)], exf)
                for i in range(G):
                    v = exf[pl.ds(i * 16, 16)]
                    m0 = v.at[jnp.full((16,), 2 * p, jnp.int32)].get(
                        mode="promise_in_bounds")
                    m1 = v.at[jnp.full((16,), 2 * p + 1, jnp.int32)].get(
                        mode="promise_in_bounds")
                    for q in range(4):
                        fcb[i, pl.ds(q * 16, 16)] = fcb[i, pl.ds(q * 16, 16)] * m0
                    for q in range(4, 8):
                        fcb[i, pl.ds(q * 16, 16)] = fcb[i, pl.ds(q * 16, 16)] * m1
                pltpu.sync_copy(fcb, rst_sp.at[dstb], add=True)
                return carry

            lax.fori_loop(0, GPT2, body, 0)
            plsc.subcore_barrier()
            for q in range(10):
                base = s * zrows + q * sch
                pltpu.sync_copy(rst_sp.at[pl.ds(base, sch)], stage)
                pltpu.sync_copy(stage, rstP_hbm.at[pl.ds(p * NV + base, sch)])

    return s2


def kernel(feat, edge_index, W_fc, attn_l, attn_r, bias_gat, W3, b3):
    N, IN_FEATS = feat.shape
    H, HF = attn_l.shape
    D = H * HF
    NP = D // LW
    NUM_CLASSES = W3.shape[1]
    E = edge_index.shape[1]

    RB = 1024
    NV = -(-(N + 1) // RB) * RB
    EG = G * NSC * NTILES
    E_pad = -(-E // EG) * EG
    NG = E_pad // G
    GPT1 = NG // (NSC * NTILES)
    GPT2 = NG // NTILES

    eye = jnp.eye(H, dtype=jnp.float32)
    A_l = jnp.concatenate(
        [(attn_l[:, :, None] * eye[:, None, :]).reshape(D, H),
         jnp.zeros((D, LW - H), jnp.float32)], axis=1)
    A_r = jnp.concatenate(
        [(attn_r[:, :, None] * eye[:, None, :]).reshape(D, H),
         jnp.zeros((D, LW - H), jnp.float32)], axis=1)
    B_bcast = jnp.repeat(eye, HF, axis=1)

    feat_p = jnp.concatenate(
        [feat, jnp.zeros((NV - N, IN_FEATS), jnp.float32)], axis=0)
    pad = jnp.full((E_pad - E,), N, jnp.int32)
    src_p = jnp.concatenate([edge_index[0].astype(jnp.int32), pad])
    dst_p = jnp.concatenate([edge_index[1].astype(jnp.int32), pad])
    z16 = jnp.zeros((NV // 2 // NTILES // 4, LW), jnp.float32)
    z128 = jnp.zeros((NV // NTILES // 10, LW), jnp.float32)

    n_rb = NV // RB
    fcP, el, er = pl.pallas_call(
        functools.partial(_k1_body, NP=NP),
        grid=(n_rb,),
        in_specs=[
            pl.BlockSpec((RB, IN_FEATS), lambda i: (i, 0)),
            pl.BlockSpec((IN_FEATS, D), lambda i: (0, 0)),
            pl.BlockSpec((D, LW), lambda i: (0, 0)),
            pl.BlockSpec((D, LW), lambda i: (0, 0)),
        ],
        out_specs=[
            pl.BlockSpec((NP, RB, LW), lambda i: (0, i, 0)),
            pl.BlockSpec((RB, LW), lambda i: (i, 0)),
            pl.BlockSpec((RB, LW), lambda i: (i, 0)),
        ],
        out_shape=[
            jax.ShapeDtypeStruct((NP, NV, LW), jnp.float32),
            jax.ShapeDtypeStruct((NV, LW), jnp.float32),
            jax.ShapeDtypeStruct((NV, LW), jnp.float32),
        ],
    )(feat_p, W_fc, A_l, A_r)

    eexp, dpart = _make_s1(NV, NG)(src_p, dst_p, el, er, z16)

    srcP = (src_p[None, :] + (jnp.arange(NP, dtype=jnp.int32) * NV)[:, None]).reshape(-1)
    rstP = _make_s2(NV, NG, GPT2, NP)(
        srcP, dst_p, fcP.reshape(NP * NV, LW), eexp, z128)

    out = pl.pallas_call(
        functools.partial(_k3_body, H=H, NP=NP),
        grid=(n_rb,),
        in_specs=[
            pl.BlockSpec((NP, RB, LW), lambda i: (0, i, 0)),
            pl.BlockSpec((RB, LW), lambda i: (i, 0)),
            pl.BlockSpec((H, D), lambda i: (0, 0)),
            pl.BlockSpec((1, D), lambda i: (0, 0)),
            pl.BlockSpec((D, NUM_CLASSES), lambda i: (0, 0)),
            pl.BlockSpec((1, NUM_CLASSES), lambda i: (0, 0)),
        ],
        out_specs=pl.BlockSpec((RB, NUM_CLASSES), lambda i: (i, 0)),
        out_shape=jax.ShapeDtypeStruct((NV, NUM_CLASSES), jnp.float32),
    )(rstP.reshape(NP, NV, LW), dpart, B_bcast,
      bias_gat.reshape(1, D), W3, b3.reshape(1, NUM_CLASSES))

    return out[:N]


# speedup vs baseline: 14.9576x; 1.1335x over previous
import functools
import jax
import jax.numpy as jnp
from jax import lax
from jax.experimental import pallas as pl
from jax.experimental.pallas import tpu as pltpu
from jax.experimental.pallas import tpu_sc as plsc

G = 128
NSC = 2
NTILES = 16
LW = 128


def _k1_body(feat_ref, wfc_ref, al_ref, ar_ref, fcP_ref, el_ref, er_ref, *, NP):
    fc = jnp.dot(feat_ref[...], wfc_ref[...], preferred_element_type=jnp.float32)
    for p in range(NP):
        fcP_ref[p] = fc[:, p * LW:(p + 1) * LW]
    el_ref[...] = jnp.dot(fc, al_ref[...], preferred_element_type=jnp.float32)
    er_ref[...] = jnp.dot(fc, ar_ref[...], preferred_element_type=jnp.float32)


def _k3_body(rstP_ref, dpart_ref, bb_ref, bias_ref, w3_ref, b3_ref, out_ref, *, H, NP):
    d = dpart_ref[:, 0:H]
    invd = 1.0 / (d + 1e-9)
    invb = jnp.dot(invd, bb_ref[...], preferred_element_type=jnp.float32)
    rst = jnp.concatenate([rstP_ref[p] for p in range(NP)], axis=-1)
    hact = rst * invb + bias_ref[...]
    hact = jnp.where(hact > 0, hact, jnp.exp(jnp.minimum(hact, 0.0)) - 1.0)
    out_ref[...] = jnp.dot(hact, w3_ref[...], preferred_element_type=jnp.float32) + b3_ref[...]


def _make_s1(NV, NG):
    mesh = plsc.VectorSubcoreMesh(core_axis_name="c", subcore_axis_name="s")
    NVH = NV // 2
    GPT = NG // NTILES
    zrows = NVH // NTILES
    sch = zrows // 10

    @functools.partial(
        pl.kernel, mesh=mesh,
        out_type=[
            jax.ShapeDtypeStruct((NG * G * 16,), jnp.float32),
            jax.ShapeDtypeStruct((NV, 128), jnp.float32),
        ],
        scratch_types=[
            pltpu.VMEM((G,), jnp.int32),
            pltpu.VMEM((G,), jnp.int32),
            pltpu.VMEM((G,), jnp.int32),
            pltpu.VMEM((G,), jnp.int32),
            pltpu.VMEM((G,), jnp.int32),
            pltpu.VMEM((G,), jnp.int32),
            pltpu.VMEM((G, LW), jnp.float32),
            pltpu.VMEM((G, LW), jnp.float32),
            pltpu.VMEM((G, LW), jnp.float32),
            pltpu.VMEM((G, LW), jnp.float32),
            pltpu.VMEM((G * 16,), jnp.float32),
            pltpu.VMEM((G * 16,), jnp.float32),
            pltpu.VMEM((NVH // NTILES // 10, LW), jnp.float32),
            pltpu.VMEM_SHARED((NVH + 16, LW), jnp.float32),
            pltpu.SemaphoreType.DMA,
            pltpu.SemaphoreType.DMA,
            pltpu.SemaphoreType.DMA,
            pltpu.SemaphoreType.DMA,
            pltpu.SemaphoreType.DMA,
            pltpu.SemaphoreType.DMA,
            pltpu.SemaphoreType.DMA,
        ],
    )
    def s1(src_hbm, dst_hbm, el_hbm, er_hbm, z_hbm, eexp_hbm, denom_hbm,
           srcb0, srcb1, dstb0, dstb1, dstc0, dstc1,
           rs0, rs1, rd0, rd1, exf0, exf1, stage, denom_sp,
           seml0, seml1, semg0, semg1, semw, sems0, sems1):
        c = lax.axis_index("c")
        s = lax.axis_index("s")
        nbase = c * NVH
        pltpu.sync_copy(z_hbm, stage)
        for q in range(10):
            pltpu.sync_copy(stage, denom_sp.at[pl.ds(s * zrows + q * sch, sch)])
        plsc.subcore_barrier()
        bufs = ((srcb0, dstb0, dstc0, rs0, rd0, exf0, seml0, semg0),
                (srcb1, dstb1, dstc1, rs1, rd1, exf1, seml1, semg1))

        def issue_loads(b, g):
            srcb, dstb, dstc, rs, rd, exf, seml, semg = bufs[b]
            h1 = pltpu.async_copy(src_hbm.at[pl.ds(g * G, G)], srcb, seml)
            h2 = pltpu.async_copy(dst_hbm.at[pl.ds(g * G, G)], dstb, seml)
            return (h1, h2)

        def issue_gathers(b):
            srcb, dstb, dstc, rs, rd, exf, seml, semg = bufs[b]
            h1 = pltpu.async_copy(el_hbm.at[srcb], rs, semg)
            h2 = pltpu.async_copy(er_hbm.at[dstb], rd, semg)
            return (h1, h2)

        def compute(b):
            srcb, dstb, dstc, rs, rd, exf, seml, semg = bufs[b]
            for k in range(G // 16):
                dv = dstb[pl.ds(k * 16, 16)] - nbase
                ok = jnp.logical_and(dv >= 0, dv < NVH)
                dstc[pl.ds(k * 16, 16)] = jnp.where(ok, dv, NVH)

            def inner(i, cc):
                v = rs[i, 0:16] + rd[i, 0:16]
                v = jnp.where(v > 0, v, 0.2 * v)
                ex = jnp.exp(v)
                rs[i, 0:16] = ex
                exf[pl.ds(i * 16, 16)] = ex
                return cc

            lax.fori_loop(0, G, inner, 0)

        def body(jj, carry):
            j0 = 2 * jj
            j1 = 2 * jj + 1
            g0i = s * GPT + j0
            g1i = s * GPT + j1
            l0 = issue_loads(0, g0i)
            l1 = issue_loads(1, g1i)
            for h in l0:
                h.wait()
            ga0 = issue_gathers(0)
            for h in l1:
                h.wait()
            ga1 = issue_gathers(1)
            for h in ga0:
                h.wait()
            compute(0)

            @pl.when(c == 0)
            def _():
                pltpu.async_copy(
                    exf0, eexp_hbm.at[pl.ds(g0i * G * 16, G * 16)], semw).wait()

            sc0 = pltpu.async_copy(rs0, denom_sp.at[dstc0], sems0, add=True)
            for h in ga1:
                h.wait()
            compute(1)

            @pl.when(c == 0)
            def _():
                pltpu.async_copy(
                    exf1, eexp_hbm.at[pl.ds(g1i * G * 16, G * 16)], semw).wait()

            sc1 = pltpu.async_copy(rs1, denom_sp.at[dstc1], sems1, add=True)
            sc0.wait()
            sc1.wait()
            return carry

        lax.fori_loop(0, GPT // 2, body, 0)
        plsc.subcore_barrier()
        for q in range(10):
            base = s * zrows + q * sch
            pltpu.sync_copy(denom_sp.at[pl.ds(base, sch)], stage)
            pltpu.sync_copy(stage, denom_hbm.at[pl.ds(nbase + base, sch)])

    return s1


def _make_s2(NV, NG, GPT2, NP):
    mesh = plsc.VectorSubcoreMesh(core_axis_name="c", subcore_axis_name="s")
    zrows = NV // NTILES
    sch = zrows // 10
    ppc = NP // NSC

    @functools.partial(
        pl.kernel, mesh=mesh,
        out_type=jax.ShapeDtypeStruct((NP * NV, LW), jnp.float32),
        scratch_types=[
            pltpu.VMEM((G,), jnp.int32),
            pltpu.VMEM((G,), jnp.int32),
            pltpu.VMEM((G,), jnp.int32),
            pltpu.VMEM((G,), jnp.int32),
            pltpu.VMEM((G * 16,), jnp.float32),
            pltpu.VMEM((G * 16,), jnp.float32),
            pltpu.VMEM((G, LW), jnp.float32),
            pltpu.VMEM((G, LW), jnp.float32),
            pltpu.VMEM((NV // NTILES // 10, LW), jnp.float32),
            pltpu.VMEM_SHARED((NV, LW), jnp.float32),
            pltpu.SemaphoreType.DMA,
            pltpu.SemaphoreType.DMA,
            pltpu.SemaphoreType.DMA,
            pltpu.SemaphoreType.DMA,
            pltpu.SemaphoreType.DMA,
            pltpu.SemaphoreType.DMA,
        ],
    )
    def s2(srcP_hbm, dst_hbm, fcP_hbm, eexp_hbm, z128_hbm, rstP_hbm,
           idxb0, idxb1, dstb0, dstb1, exf0, exf1, fcb0, fcb1, stage, rst_sp,
           seml0, seml1, semg0, semg1, sems0, sems1):
        c = lax.axis_index("c")
        s = lax.axis_index("s")
        bufs = ((idxb0, dstb0, exf0, fcb0, seml0, semg0, sems0),
                (idxb1, dstb1, exf1, fcb1, seml1, semg1, sems1))
        for pp in range(ppc):
            p = c * ppc + pp
            pltpu.sync_copy(z128_hbm, stage)
            for q in range(10):
                pltpu.sync_copy(
                    stage, rst_sp.at[pl.ds(s * zrows + q * sch, sch)])
            plsc.subcore_barrier()

            def issue_loads(b, g):
                idxb, dstb, exf, fcb, seml, semg, sems = bufs[b]
                h1 = pltpu.async_copy(
                    srcP_hbm.at[pl.ds(p * NG * G + g * G, G)], idxb, seml)
                h2 = pltpu.async_copy(dst_hbm.at[pl.ds(g * G, G)], dstb, seml)
                h3 = pltpu.async_copy(
                    eexp_hbm.at[pl.ds(g * G * 16, G * 16)], exf, seml)
                return (h1, h2, h3)

            def compute(b):
                idxb, dstb, exf, fcb, seml, semg, sems = bufs[b]
                for i in range(G):
                    v = exf[pl.ds(i * 16, 16)]
                    m0 = v.at[jnp.full((16,), 2 * p, jnp.int32)].get(
                        mode="promise_in_bounds")
                    m1 = v.at[jnp.full((16,), 2 * p + 1, jnp.int32)].get(
                        mode="promise_in_bounds")
                    for q in range(4):
                        fcb[i, pl.ds(q * 16, 16)] = fcb[i, pl.ds(q * 16, 16)] * m0
                    for q in range(4, 8):
                        fcb[i, pl.ds(q * 16, 16)] = fcb[i, pl.ds(q * 16, 16)] * m1

            def body(jj, carry):
                j0 = 2 * jj
                j1 = 2 * jj + 1
                l0 = issue_loads(0, s * GPT2 + j0)
                l1 = issue_loads(1, s * GPT2 + j1)
                for h in l0:
                    h.wait()
                g0 = pltpu.async_copy(fcP_hbm.at[idxb0], fcb0, semg0)
                for h in l1:
                    h.wait()
                g1 = pltpu.async_copy(fcP_hbm.at[idxb1], fcb1, semg1)
                g0.wait()
                compute(0)
                s0 = pltpu.async_copy(fcb0, rst_sp.at[dstb0], sems0, add=True)
                g1.wait()
                compute(1)
                s1 = pltpu.async_copy(fcb1, rst_sp.at[dstb1], sems1, add=True)
                s0.wait()
                s1.wait()
                return carry

            lax.fori_loop(0, GPT2 // 2, body, 0)
            plsc.subcore_barrier()
            for q in range(10):
                base = s * zrows + q * sch
                pltpu.sync_copy(rst_sp.at[pl.ds(base, sch)], stage)
                pltpu.sync_copy(stage, rstP_hbm.at[pl.ds(p * NV + base, sch)])

    return s2


def kernel(feat, edge_index, W_fc, attn_l, attn_r, bias_gat, W3, b3):
    N, IN_FEATS = feat.shape
    H, HF = attn_l.shape
    D = H * HF
    NP = D // LW
    NUM_CLASSES = W3.shape[1]
    E = edge_index.shape[1]

    RB = 1024
    NV = -(-(N + 1) // RB) * RB
    EG = G * NSC * NTILES
    E_pad = -(-E // EG) * EG
    NG = E_pad // G
    GPT1 = NG // (NSC * NTILES)
    GPT2 = NG // NTILES

    eye = jnp.eye(H, dtype=jnp.float32)
    A_l = jnp.concatenate(
        [(attn_l[:, :, None] * eye[:, None, :]).reshape(D, H),
         jnp.zeros((D, LW - H), jnp.float32)], axis=1)
    A_r = jnp.concatenate(
        [(attn_r[:, :, None] * eye[:, None, :]).reshape(D, H),
         jnp.zeros((D, LW - H), jnp.float32)], axis=1)
    B_bcast = jnp.repeat(eye, HF, axis=1)

    feat_p = jnp.concatenate(
        [feat, jnp.zeros((NV - N, IN_FEATS), jnp.float32)], axis=0)
    pad = jnp.full((E_pad - E,), N, jnp.int32)
    src_p = jnp.concatenate([edge_index[0].astype(jnp.int32), pad])
    dst_p = jnp.concatenate([edge_index[1].astype(jnp.int32), pad])
    z16 = jnp.zeros((NV // 2 // NTILES // 10, LW), jnp.float32)
    z128 = jnp.zeros((NV // NTILES // 10, LW), jnp.float32)

    n_rb = NV // RB
    fcP, el, er = pl.pallas_call(
        functools.partial(_k1_body, NP=NP),
        grid=(n_rb,),
        in_specs=[
            pl.BlockSpec((RB, IN_FEATS), lambda i: (i, 0)),
            pl.BlockSpec((IN_FEATS, D), lambda i: (0, 0)),
            pl.BlockSpec((D, LW), lambda i: (0, 0)),
            pl.BlockSpec((D, LW), lambda i: (0, 0)),
        ],
        out_specs=[
            pl.BlockSpec((NP, RB, LW), lambda i: (0, i, 0)),
            pl.BlockSpec((RB, LW), lambda i: (i, 0)),
            pl.BlockSpec((RB, LW), lambda i: (i, 0)),
        ],
        out_shape=[
            jax.ShapeDtypeStruct((NP, NV, LW), jnp.float32),
            jax.ShapeDtypeStruct((NV, LW), jnp.float32),
            jax.ShapeDtypeStruct((NV, LW), jnp.float32),
        ],
    )(feat_p, W_fc, A_l, A_r)

    eexp, dpart = _make_s1(NV, NG)(src_p, dst_p, el, er, z16)

    srcP = (src_p[None, :] + (jnp.arange(NP, dtype=jnp.int32) * NV)[:, None]).reshape(-1)
    rstP = _make_s2(NV, NG, GPT2, NP)(
        srcP, dst_p, fcP.reshape(NP * NV, LW), eexp, z128)

    out = pl.pallas_call(
        functools.partial(_k3_body, H=H, NP=NP),
        grid=(n_rb,),
        in_specs=[
            pl.BlockSpec((NP, RB, LW), lambda i: (0, i, 0)),
            pl.BlockSpec((RB, LW), lambda i: (i, 0)),
            pl.BlockSpec((H, D), lambda i: (0, 0)),
            pl.BlockSpec((1, D), lambda i: (0, 0)),
            pl.BlockSpec((D, NUM_CLASSES), lambda i: (0, 0)),
            pl.BlockSpec((1, NUM_CLASSES), lambda i: (0, 0)),
        ],
        out_specs=pl.BlockSpec((RB, NUM_CLASSES), lambda i: (i, 0)),
        out_shape=jax.ShapeDtypeStruct((NV, NUM_CLASSES), jnp.float32),
    )(rstP.reshape(NP, NV, LW), dpart, B_bcast,
      bias_gat.reshape(1, D), W3, b3.reshape(1, NUM_CLASSES))

    return out[:N]


# speedup vs baseline: 16.8964x; 1.1296x over previous
import functools
import jax
import jax.numpy as jnp
from jax import lax
from jax.experimental import pallas as pl
from jax.experimental.pallas import tpu as pltpu
from jax.experimental.pallas import tpu_sc as plsc

G = 128
NSC = 2
NTILES = 16
LW = 128


def _k1_body(feat_ref, wfc_ref, al_ref, ar_ref, fcP_ref, el_ref, er_ref, *, NP):
    fc = jnp.dot(feat_ref[...], wfc_ref[...], preferred_element_type=jnp.float32)
    for p in range(NP):
        fcP_ref[p] = fc[:, p * LW:(p + 1) * LW]
    el_ref[...] = jnp.dot(fc, al_ref[...], preferred_element_type=jnp.float32)
    er_ref[...] = jnp.dot(fc, ar_ref[...], preferred_element_type=jnp.float32)


def _k3_body(rstP_ref, dpart_ref, bb_ref, bias_ref, w3_ref, b3_ref, out_ref, *, H, NP):
    d = dpart_ref[0, :, 0:H] + dpart_ref[1, :, 0:H]
    invd = 1.0 / (d + 1e-9)
    invb = jnp.dot(invd, bb_ref[...], preferred_element_type=jnp.float32)
    rst = jnp.concatenate([rstP_ref[p] for p in range(NP)], axis=-1)
    hact = rst * invb + bias_ref[...]
    hact = jnp.where(hact > 0, hact, jnp.exp(jnp.minimum(hact, 0.0)) - 1.0)
    out_ref[...] = jnp.dot(hact, w3_ref[...], preferred_element_type=jnp.float32) + b3_ref[...]


def _make_s1(NV, NG):
    mesh = plsc.VectorSubcoreMesh(core_axis_name="c", subcore_axis_name="s")
    GPT = NG // (NSC * NTILES)
    zrows = NV // NTILES
    sch = zrows // 10

    @functools.partial(
        pl.kernel, mesh=mesh,
        out_type=[
            jax.ShapeDtypeStruct((NG * G * 16,), jnp.float32),
            jax.ShapeDtypeStruct((NSC * NV, 128), jnp.float32),
        ],
        scratch_types=[
            pltpu.VMEM((G,), jnp.int32),
            pltpu.VMEM((G,), jnp.int32),
            pltpu.VMEM((G, LW), jnp.float32),
            pltpu.VMEM((G, LW), jnp.float32),
            pltpu.VMEM((G * 16,), jnp.float32),
            pltpu.VMEM((NV // NTILES // 10, LW), jnp.float32),
            pltpu.VMEM_SHARED((NV, LW), jnp.float32),
            pltpu.SemaphoreType.DMA,
            pltpu.SemaphoreType.DMA,
            pltpu.SemaphoreType.DMA,
            pltpu.SemaphoreType.DMA,
        ],
    )
    def s1(src_hbm, dst_hbm, el_hbm, er_hbm, z_hbm, eexp_hbm, dpart_hbm,
           srcb, dstb, rs, rd, exf, stage, denom_sp, seml, semg, semw, sems):
        c = lax.axis_index("c")
        s = lax.axis_index("s")
        wid = c * NTILES + s
        pltpu.sync_copy(z_hbm, stage)
        for q in range(10):
            pltpu.sync_copy(stage, denom_sp.at[pl.ds(s * zrows + q * sch, sch)])
        plsc.subcore_barrier()

        def body(j, carry):
            g = wid * GPT + j
            h1 = pltpu.async_copy(src_hbm.at[pl.ds(g * G, G)], srcb, seml)
            h2 = pltpu.async_copy(dst_hbm.at[pl.ds(g * G, G)], dstb, seml)
            h1.wait()
            h2.wait()
            ga = pltpu.async_copy(el_hbm.at[srcb], rs, semg)
            gb = pltpu.async_copy(er_hbm.at[dstb], rd, semg)
            ga.wait()
            gb.wait()

            def inner(i, cc):
                v = rs[i, 0:16] + rd[i, 0:16]
                v = jnp.where(v > 0, v, 0.2 * v)
                ex = jnp.exp(v)
                rs[i, 0:16] = ex
                exf[pl.ds(i * 16, 16)] = ex
                return cc

            lax.fori_loop(0, G, inner, 0)
            w = pltpu.async_copy(exf, eexp_hbm.at[pl.ds(g * G * 16, G * 16)], semw)
            sc = pltpu.async_copy(rs, denom_sp.at[dstb], sems, add=True)
            w.wait()
            sc.wait()
            return carry

        lax.fori_loop(0, GPT, body, 0)
        plsc.subcore_barrier()
        for q in range(10):
            base = s * zrows + q * sch
            pltpu.sync_copy(denom_sp.at[pl.ds(base, sch)], stage)
            pltpu.sync_copy(stage, dpart_hbm.at[pl.ds(c * NV + base, sch)])

    return s1


def _make_s2(NV, NG, GPT2, NP):
    mesh = plsc.VectorSubcoreMesh(core_axis_name="c", subcore_axis_name="s")
    zrows = NV // NTILES
    sch = zrows // 10
    ppc = NP // NSC

    @functools.partial(
        pl.kernel, mesh=mesh,
        out_type=jax.ShapeDtypeStruct((NP * NV, LW), jnp.float32),
        scratch_types=[
            pltpu.VMEM((G,), jnp.int32),
            pltpu.VMEM((G,), jnp.int32),
            pltpu.VMEM((G,), jnp.int32),
            pltpu.VMEM((G,), jnp.int32),
            pltpu.VMEM((G * 16,), jnp.float32),
            pltpu.VMEM((G * 16,), jnp.float32),
            pltpu.VMEM((G, LW), jnp.float32),
            pltpu.VMEM((G, LW), jnp.float32),
            pltpu.VMEM((NV // NTILES // 10, LW), jnp.float32),
            pltpu.VMEM_SHARED((NV, LW), jnp.float32),
            pltpu.SemaphoreType.DMA,
            pltpu.SemaphoreType.DMA,
            pltpu.SemaphoreType.DMA,
            pltpu.SemaphoreType.DMA,
            pltpu.SemaphoreType.DMA,
            pltpu.SemaphoreType.DMA,
        ],
    )
    def s2(srcP_hbm, dst_hbm, fcP_hbm, eexp_hbm, z128_hbm, rstP_hbm,
           idxb0, idxb1, dstb0, dstb1, exf0, exf1, fcb0, fcb1, stage, rst_sp,
           seml0, seml1, semg0, semg1, sems0, sems1):
        c = lax.axis_index("c")
        s = lax.axis_index("s")
        bufs = ((idxb0, dstb0, exf0, fcb0, seml0, semg0, sems0),
                (idxb1, dstb1, exf1, fcb1, seml1, semg1, sems1))
        for pp in range(ppc):
            p = c * ppc + pp
            pltpu.sync_copy(z128_hbm, stage)
            for q in range(10):
                pltpu.sync_copy(
                    stage, rst_sp.at[pl.ds(s * zrows + q * sch, sch)])
            plsc.subcore_barrier()

            def issue_loads(b, g):
                idxb, dstb, exf, fcb, seml, semg, sems = bufs[b]
                h1 = pltpu.async_copy(
                    srcP_hbm.at[pl.ds(p * NG * G + g * G, G)], idxb, seml)
                h2 = pltpu.async_copy(dst_hbm.at[pl.ds(g * G, G)], dstb, seml)
                h3 = pltpu.async_copy(
                    eexp_hbm.at[pl.ds(g * G * 16, G * 16)], exf, seml)
                return (h1, h2, h3)

            def compute(b):
                idxb, dstb, exf, fcb, seml, semg, sems = bufs[b]
                for i in range(G):
                    v = exf[pl.ds(i * 16, 16)]
                    m0 = v.at[jnp.full((16,), 2 * p, jnp.int32)].get(
                        mode="promise_in_bounds")
                    m1 = v.at[jnp.full((16,), 2 * p + 1, jnp.int32)].get(
                        mode="promise_in_bounds")
                    for q in range(4):
                        fcb[i, pl.ds(q * 16, 16)] = fcb[i, pl.ds(q * 16, 16)] * m0
                    for q in range(4, 8):
                        fcb[i, pl.ds(q * 16, 16)] = fcb[i, pl.ds(q * 16, 16)] * m1

            def body(jj, carry):
                j0 = 2 * jj
                j1 = 2 * jj + 1
                l0 = issue_loads(0, s * GPT2 + j0)
                l1 = issue_loads(1, s * GPT2 + j1)
                for h in l0:
                    h.wait()
                g0 = pltpu.async_copy(fcP_hbm.at[idxb0], fcb0, semg0)
                for h in l1:
                    h.wait()
                g1 = pltpu.async_copy(fcP_hbm.at[idxb1], fcb1, semg1)
                g0.wait()
                compute(0)
                s0 = pltpu.async_copy(fcb0, rst_sp.at[dstb0], sems0, add=True)
                g1.wait()
                compute(1)
                s1 = pltpu.async_copy(fcb1, rst_sp.at[dstb1], sems1, add=True)
                s0.wait()
                s1.wait()
                return carry

            lax.fori_loop(0, GPT2 // 2, body, 0)
            plsc.subcore_barrier()
            for q in range(10):
                base = s * zrows + q * sch
                pltpu.sync_copy(rst_sp.at[pl.ds(base, sch)], stage)
                pltpu.sync_copy(stage, rstP_hbm.at[pl.ds(p * NV + base, sch)])

    return s2


def kernel(feat, edge_index, W_fc, attn_l, attn_r, bias_gat, W3, b3):
    N, IN_FEATS = feat.shape
    H, HF = attn_l.shape
    D = H * HF
    NP = D // LW
    NUM_CLASSES = W3.shape[1]
    E = edge_index.shape[1]

    RB = 1024
    NV = -(-(N + 1) // RB) * RB
    EG = G * NSC * NTILES
    E_pad = -(-E // EG) * EG
    NG = E_pad // G
    GPT1 = NG // (NSC * NTILES)
    GPT2 = NG // NTILES

    eye = jnp.eye(H, dtype=jnp.float32)
    A_l = jnp.concatenate(
        [(attn_l[:, :, None] * eye[:, None, :]).reshape(D, H),
         jnp.zeros((D, LW - H), jnp.float32)], axis=1)
    A_r = jnp.concatenate(
        [(attn_r[:, :, None] * eye[:, None, :]).reshape(D, H),
         jnp.zeros((D, LW - H), jnp.float32)], axis=1)
    B_bcast = jnp.repeat(eye, HF, axis=1)

    feat_p = jnp.concatenate(
        [feat, jnp.zeros((NV - N, IN_FEATS), jnp.float32)], axis=0)
    pad = jnp.full((E_pad - E,), N, jnp.int32)
    src_p = jnp.concatenate([edge_index[0].astype(jnp.int32), pad])
    dst_p = jnp.concatenate([edge_index[1].astype(jnp.int32), pad])
    z16 = jnp.zeros((NV // NTILES // 10, LW), jnp.float32)
    z128 = jnp.zeros((NV // NTILES // 10, LW), jnp.float32)

    n_rb = NV // RB
    fcP, el, er = pl.pallas_call(
        functools.partial(_k1_body, NP=NP),
        grid=(n_rb,),
        in_specs=[
            pl.BlockSpec((RB, IN_FEATS), lambda i: (i, 0)),
            pl.BlockSpec((IN_FEATS, D), lambda i: (0, 0)),
            pl.BlockSpec((D, LW), lambda i: (0, 0)),
            pl.BlockSpec((D, LW), lambda i: (0, 0)),
        ],
        out_specs=[
            pl.BlockSpec((NP, RB, LW), lambda i: (0, i, 0)),
            pl.BlockSpec((RB, LW), lambda i: (i, 0)),
            pl.BlockSpec((RB, LW), lambda i: (i, 0)),
        ],
        out_shape=[
            jax.ShapeDtypeStruct((NP, NV, LW), jnp.float32),
            jax.ShapeDtypeStruct((NV, LW), jnp.float32),
            jax.ShapeDtypeStruct((NV, LW), jnp.float32),
        ],
    )(feat_p, W_fc, A_l, A_r)

    eexp, dpart = _make_s1(NV, NG)(src_p, dst_p, el, er, z16)

    srcP = (src_p[None, :] + (jnp.arange(NP, dtype=jnp.int32) * NV)[:, None]).reshape(-1)
    rstP = _make_s2(NV, NG, GPT2, NP)(
        srcP, dst_p, fcP.reshape(NP * NV, LW), eexp, z128)

    out = pl.pallas_call(
        functools.partial(_k3_body, H=H, NP=NP),
        grid=(n_rb,),
        in_specs=[
            pl.BlockSpec((NP, RB, LW), lambda i: (0, i, 0)),
            pl.BlockSpec((NSC, RB, LW), lambda i: (0, i, 0)),
            pl.BlockSpec((H, D), lambda i: (0, 0)),
            pl.BlockSpec((1, D), lambda i: (0, 0)),
            pl.BlockSpec((D, NUM_CLASSES), lambda i: (0, 0)),
            pl.BlockSpec((1, NUM_CLASSES), lambda i: (0, 0)),
        ],
        out_specs=pl.BlockSpec((RB, NUM_CLASSES), lambda i: (i, 0)),
        out_shape=jax.ShapeDtypeStruct((NV, NUM_CLASSES), jnp.float32),
    )(rstP.reshape(NP, NV, LW), dpart.reshape(NSC, NV, LW), B_bcast,
      bias_gat.reshape(1, D), W3, b3.reshape(1, NUM_CLASSES))

    return out[:N]


# speedup vs baseline: 16.9010x; 1.0003x over previous
import functools
import jax
import jax.numpy as jnp
from jax import lax
from jax.experimental import pallas as pl
from jax.experimental.pallas import tpu as pltpu
from jax.experimental.pallas import tpu_sc as plsc

G = 128
NSC = 2
NTILES = 16
LW = 128


def _k1_body(feat_ref, wfc_ref, al_ref, ar_ref, fcP_ref, el_ref, er_ref, *, NP):
    fc = jnp.dot(feat_ref[...], wfc_ref[...], preferred_element_type=jnp.float32)
    for p in range(NP):
        fcP_ref[p] = fc[:, p * LW:(p + 1) * LW]
    el_ref[...] = jnp.dot(fc, al_ref[...], preferred_element_type=jnp.float32)
    er_ref[...] = jnp.dot(fc, ar_ref[...], preferred_element_type=jnp.float32)


def _k3_body(rstP_ref, dpart_ref, bb_ref, bias_ref, w3_ref, b3_ref, out_ref, *, H, NP):
    d = dpart_ref[0, :, 0:H] + dpart_ref[1, :, 0:H]
    invd = 1.0 / (d + 1e-9)
    invb = jnp.dot(invd, bb_ref[...], preferred_element_type=jnp.float32)
    rst = jnp.concatenate([rstP_ref[p] for p in range(NP)], axis=-1)
    hact = rst * invb + bias_ref[...]
    hact = jnp.where(hact > 0, hact, jnp.exp(jnp.minimum(hact, 0.0)) - 1.0)
    out_ref[...] = jnp.dot(hact, w3_ref[...], preferred_element_type=jnp.float32) + b3_ref[...]


def _make_s1(NV, NG):
    mesh = plsc.VectorSubcoreMesh(core_axis_name="c", subcore_axis_name="s")
    GPT = NG // (NSC * NTILES)
    zrows = NV // NTILES
    sch = zrows // 10

    @functools.partial(
        pl.kernel, mesh=mesh,
        out_type=[
            jax.ShapeDtypeStruct((NG * G * 16,), jnp.float32),
            jax.ShapeDtypeStruct((NSC * NV, 128), jnp.float32),
        ],
        scratch_types=[
            pltpu.VMEM((G,), jnp.int32),
            pltpu.VMEM((G,), jnp.int32),
            pltpu.VMEM((G, LW), jnp.float32),
            pltpu.VMEM((G, LW), jnp.float32),
            pltpu.VMEM((G * 16,), jnp.float32),
            pltpu.VMEM((NV // NTILES // 10, LW), jnp.float32),
            pltpu.VMEM_SHARED((NV, LW), jnp.float32),
            pltpu.SemaphoreType.DMA,
            pltpu.SemaphoreType.DMA,
            pltpu.SemaphoreType.DMA,
            pltpu.SemaphoreType.DMA,
        ],
    )
    def s1(src_hbm, dst_hbm, el_hbm, er_hbm, z_hbm, eexp_hbm, dpart_hbm,
           srcb, dstb, rs, rd, exf, stage, denom_sp, seml, semg, semw, sems):
        c = lax.axis_index("c")
        s = lax.axis_index("s")
        wid = c * NTILES + s
        pltpu.sync_copy(z_hbm, stage)
        for q in range(10):
            pltpu.sync_copy(stage, denom_sp.at[pl.ds(s * zrows + q * sch, sch)])
        plsc.subcore_barrier()

        def body(j, carry):
            g = wid * GPT + j
            h1 = pltpu.async_copy(src_hbm.at[pl.ds(g * G, G)], srcb, seml)
            h2 = pltpu.async_copy(dst_hbm.at[pl.ds(g * G, G)], dstb, seml)
            h1.wait()
            h2.wait()
            ga = pltpu.async_copy(el_hbm.at[srcb], rs, semg)
            gb = pltpu.async_copy(er_hbm.at[dstb], rd, semg)
            ga.wait()
            gb.wait()

            def inner(i, cc):
                v = rs[i, 0:16] + rd[i, 0:16]
                v = jnp.where(v > 0, v, 0.2 * v)
                ex = jnp.exp(v)
                rs[i, 0:16] = ex
                exf[pl.ds(i * 16, 16)] = ex
                return cc

            lax.fori_loop(0, G, inner, 0)
            w = pltpu.async_copy(exf, eexp_hbm.at[pl.ds(g * G * 16, G * 16)], semw)
            sc = pltpu.async_copy(rs, denom_sp.at[dstb], sems, add=True)
            w.wait()
            sc.wait()
            return carry

        lax.fori_loop(0, GPT, body, 0)
        plsc.subcore_barrier()
        for q in range(10):
            base = s * zrows + q * sch
            pltpu.sync_copy(denom_sp.at[pl.ds(base, sch)], stage)
            pltpu.sync_copy(stage, dpart_hbm.at[pl.ds(c * NV + base, sch)])

    return s1


def _make_s2(NV, NG, GPT2, NP):
    mesh = plsc.VectorSubcoreMesh(core_axis_name="c", subcore_axis_name="s")
    zrows = NV // NTILES
    sch = zrows // 10
    ppc = NP // NSC

    @functools.partial(
        pl.kernel, mesh=mesh,
        out_type=jax.ShapeDtypeStruct((NP * NV, LW), jnp.float32),
        scratch_types=[
            pltpu.VMEM((G,), jnp.int32),
            pltpu.VMEM((G,), jnp.int32),
            pltpu.VMEM((G,), jnp.int32),
            pltpu.VMEM((G,), jnp.int32),
            pltpu.VMEM((G * 16,), jnp.float32),
            pltpu.VMEM((G * 16,), jnp.float32),
            pltpu.VMEM((G, LW), jnp.float32),
            pltpu.VMEM((G, LW), jnp.float32),
            pltpu.VMEM((NV // NTILES // 10, LW), jnp.float32),
            pltpu.VMEM_SHARED((NV, LW), jnp.float32),
            pltpu.SemaphoreType.DMA,
            pltpu.SemaphoreType.DMA,
            pltpu.SemaphoreType.DMA,
            pltpu.SemaphoreType.DMA,
            pltpu.SemaphoreType.DMA,
            pltpu.SemaphoreType.DMA,
        ],
    )
    def s2(srcP_hbm, dst_hbm, fcP_hbm, eexp_hbm, z128_hbm, rstP_hbm,
           idxb0, idxb1, dstb0, dstb1, exf0, exf1, fcb0, fcb1, stage, rst_sp,
           seml0, seml1, semg0, semg1, sems0, sems1):
        c = lax.axis_index("c")
        s = lax.axis_index("s")
        bufs = ((idxb0, dstb0, exf0, fcb0, seml0, semg0, sems0),
                (idxb1, dstb1, exf1, fcb1, seml1, semg1, sems1))
        for pp in range(ppc):
            p = c * ppc + pp
            pltpu.sync_copy(z128_hbm, stage)
            for q in range(10):
                pltpu.sync_copy(
                    stage, rst_sp.at[pl.ds(s * zrows + q * sch, sch)])
            plsc.subcore_barrier()

            def issue_loads(b, g):
                idxb, dstb, exf, fcb, seml, semg, sems = bufs[b]
                h1 = pltpu.async_copy(
                    srcP_hbm.at[pl.ds(p * NG * G + g * G, G)], idxb, seml)
                h2 = pltpu.async_copy(dst_hbm.at[pl.ds(g * G, G)], dstb, seml)
                h3 = pltpu.async_copy(
                    eexp_hbm.at[pl.ds(g * G * 16, G * 16)], exf, seml)
                return (h1, h2, h3)

            def compute(b):
                idxb, dstb, exf, fcb, seml, semg, sems = bufs[b]
                for i in range(G):
                    v = exf[pl.ds(i * 16, 16)]
                    m0 = v.at[jnp.full((16,), 2 * p, jnp.int32)].get(
                        mode="promise_in_bounds")
                    m1 = v.at[jnp.full((16,), 2 * p + 1, jnp.int32)].get(
                        mode="promise_in_bounds")
                    for q in range(4):
                        fcb[i, pl.ds(q * 16, 16)] = fcb[i, pl.ds(q * 16, 16)] * m0
                    for q in range(4, 8):
                        fcb[i, pl.ds(q * 16, 16)] = fcb[i, pl.ds(q * 16, 16)] * m1

            def body(jj, carry):
                j0 = 2 * jj
                j1 = 2 * jj + 1
                l0 = issue_loads(0, s * GPT2 + j0)
                l1 = issue_loads(1, s * GPT2 + j1)
                for h in l0:
                    h.wait()
                g0 = pltpu.async_copy(fcP_hbm.at[idxb0], fcb0, semg0)
                for h in l1:
                    h.wait()
                g1 = pltpu.async_copy(fcP_hbm.at[idxb1], fcb1, semg1)
                g0.wait()
                compute(0)
                s0 = pltpu.async_copy(fcb0, rst_sp.at[dstb0], sems0, add=True)
                g1.wait()
                compute(1)
                s1 = pltpu.async_copy(fcb1, rst_sp.at[dstb1], sems1, add=True)
                s0.wait()
                s1.wait()
                return carry

            lax.fori_loop(0, GPT2 // 2, body, 0)
            plsc.subcore_barrier()
            for q in range(10):
                base = s * zrows + q * sch
                pltpu.sync_copy(rst_sp.at[pl.ds(base, sch)], stage)
                pltpu.sync_copy(stage, rstP_hbm.at[pl.ds(p * NV + base, sch)])

    return s2


def kernel(feat, edge_index, W_fc, attn_l, attn_r, bias_gat, W3, b3):
    N, IN_FEATS = feat.shape
    H, HF = attn_l.shape
    D = H * HF
    NP = D // LW
    NUM_CLASSES = W3.shape[1]
    E = edge_index.shape[1]

    RB = 1024
    NV = -(-(N + 1) // RB) * RB
    EG = G * NSC * NTILES
    E_pad = -(-E // EG) * EG
    NG = E_pad // G
    GPT2 = NG // NTILES

    eye = jnp.eye(H, dtype=jnp.float32)
    A_l = jnp.concatenate(
        [(attn_l[:, :, None] * eye[:, None, :]).reshape(D, H),
         jnp.zeros((D, LW - H), jnp.float32)], axis=1)
    A_r = jnp.concatenate(
        [(attn_r[:, :, None] * eye[:, None, :]).reshape(D, H),
         jnp.zeros((D, LW - H), jnp.float32)], axis=1)
    B_bcast = jnp.repeat(eye, HF, axis=1)

    feat_p = jnp.concatenate(
        [feat, jnp.zeros((NV - N, IN_FEATS), jnp.float32)], axis=0)
    pad = jnp.full((E_pad - E,), N, jnp.int32)
    src_p = jnp.concatenate([edge_index[0].astype(jnp.int32), pad])
    dst_p = jnp.concatenate([edge_index[1].astype(jnp.int32), pad])
    z16 = jnp.zeros((NV // NTILES // 10, LW), jnp.float32)
    z128 = jnp.zeros((NV // NTILES // 10, LW), jnp.float32)

    n_rb = NV // RB
    fcP, el, er = pl.pallas_call(
        functools.partial(_k1_body, NP=NP),
        grid=(n_rb,),
        in_specs=[
            pl.BlockSpec((RB, IN_FEATS), lambda i: (i, 0)),
            pl.BlockSpec((IN_FEATS, D), lambda i: (0, 0)),
            pl.BlockSpec((D, LW), lambda i: (0, 0)),
            pl.BlockSpec((D, LW), lambda i: (0, 0)),
        ],
        out_specs=[
            pl.BlockSpec((NP, RB, LW), lambda i: (0, i, 0)),
            pl.BlockSpec((RB, LW), lambda i: (i, 0)),
            pl.BlockSpec((RB, LW), lambda i: (i, 0)),
        ],
        out_shape=[
            jax.ShapeDtypeStruct((NP, NV, LW), jnp.float32),
            jax.ShapeDtypeStruct((NV, LW), jnp.float32),
            jax.ShapeDtypeStruct((NV, LW), jnp.float32),
        ],
    )(feat_p, W_fc, A_l, A_r)

    eexp, dpart = _make_s1(NV, NG)(src_p, dst_p, el, er, z16)

    srcP = (src_p[None, :] + (jnp.arange(NP, dtype=jnp.int32) * NV)[:, None]).reshape(-1)
    rstP = _make_s2(NV, NG, GPT2, NP)(
        srcP, dst_p, fcP.reshape(NP * NV, LW), eexp, z128)

    out = pl.pallas_call(
        functools.partial(_k3_body, H=H, NP=NP),
        grid=(n_rb,),
        in_specs=[
            pl.BlockSpec((NP, RB, LW), lambda i: (0, i, 0)),
            pl.BlockSpec((NSC, RB, LW), lambda i: (0, i, 0)),
            pl.BlockSpec((H, D), lambda i: (0, 0)),
            pl.BlockSpec((1, D), lambda i: (0, 0)),
            pl.BlockSpec((D, NUM_CLASSES), lambda i: (0, 0)),
            pl.BlockSpec((1, NUM_CLASSES), lambda i: (0, 0)),
        ],
        out_specs=pl.BlockSpec((RB, NUM_CLASSES), lambda i: (i, 0)),
        out_shape=jax.ShapeDtypeStruct((NV, NUM_CLASSES), jnp.float32),
    )(rstP.reshape(NP, NV, LW), dpart.reshape(NSC, NV, LW), B_bcast,
      bias_gat.reshape(1, D), W3, b3.reshape(1, NUM_CLASSES))

    return out[:N]
